# baseline (speedup 1.0000x reference)
import functools

import jax
import jax.numpy as jnp
from jax import lax
from jax.experimental import pallas as pl
from jax.experimental.pallas import tpu as pltpu
from jax.experimental.pallas import tpu_sc as plsc

N = 10000
E = 320000
F_IN = 128
H1 = 512
H2 = 1024
G = 128
C = 20

FC = 128
NCORE = 2
NSUB = 16
N_SC = 10112
ROWS_PT = N_SC // NSUB
EB = 80
EPS = 1e-5


def _sc_degree(dst, zeros128, ones128):
    mesh = plsc.VectorSubcoreMesh(core_axis_name="c", subcore_axis_name="s",
                                  num_cores=NCORE, num_subcores=NSUB)
    ept = E // (NCORE * NSUB)
    nb = ept // EB

    @functools.partial(
        pl.kernel,
        out_type=jax.ShapeDtypeStruct((NCORE, N_SC, FC), jnp.float32),
        mesh=mesh,
        scratch_types=[
            pltpu.VMEM((2, EB), jnp.int32),
            pltpu.VMEM((EB, FC), jnp.float32),
            pltpu.VMEM_SHARED((N_SC, FC), jnp.float32),
            pltpu.SemaphoreType.DMA,
            pltpu.SemaphoreType.DMA,
        ],
    )
    def deg_kernel(dst_hbm, z16_hbm, ones_hbm, deg_hbm, dst_v, ones_v,
                   acc, isem, ssem):
        cid = lax.axis_index("c")
        sid = lax.axis_index("s")
        lo = sid * ROWS_PT
        base0 = (cid * NSUB + sid) * ept
        nblk = nb // 2
        rem = nb - nblk * 2

        def idx_load(k, b):
            pltpu.async_copy(dst_hbm.at[pl.ds(base0 + k * EB, EB)],
                             dst_v.at[b], isem)

        def wait_idx(k, b):
            pltpu.make_async_copy(dst_hbm.at[pl.ds(base0 + k * EB, EB)],
                                  dst_v.at[b], isem).wait()

        def scat(b):
            pltpu.async_copy(ones_v, acc.at[dst_v.at[b]], ssem, add=True)

        def wait_scat(b):
            pltpu.make_async_copy(ones_v, acc.at[dst_v.at[b]], ssem).wait()

        pltpu.sync_copy(ones_hbm, ones_v)
        pltpu.sync_copy(z16_hbm, acc.at[pl.ds(lo, ROWS_PT)])
        plsc.subcore_barrier()

        for b in range(2):
            idx_load(b, b)

        @pl.loop(0, nblk - 1)
        def _(j):
            base = j * 2
            for b in range(2):
                wait_idx(base + b, b)
                scat(b)
            for b in range(2):
                wait_scat(b)
                idx_load(base + 2 + b, b)

        for b in range(2):
            wait_idx((nblk - 1) * 2 + b, b)
            scat(b)
        for b in range(2):
            wait_scat(b)
        for r in range(rem):
            k = nblk * 2 + r
            idx_load(k, r)
            wait_idx(k, r)
            pltpu.sync_copy(ones_v, acc.at[dst_v.at[r]], add=True)

        plsc.subcore_barrier()
        pltpu.sync_copy(acc.at[pl.ds(lo, ROWS_PT)],
                        deg_hbm.at[cid].at[pl.ds(lo, ROWS_PT)])

    return deg_kernel(dst, zeros128, ones128)


EBA = 96
NSLOT = 4


def _sc_aggregate(nchunk, y, src, dst, zeros128):
    mesh = plsc.VectorSubcoreMesh(core_axis_name="c", subcore_axis_name="s",
                                  num_cores=NCORE, num_subcores=NSUB)
    cpc = nchunk // NCORE
    ept = -(-E // (NSUB * EBA)) * EBA
    pad = ept * NSUB - E
    src = jnp.concatenate([src, jnp.zeros((pad,), jnp.int32)])
    dst = jnp.concatenate([dst, jnp.full((pad,), N, jnp.int32)])
    nbt = ept // EBA
    nblk = nbt // NSLOT
    rem = nbt - nblk * NSLOT

    @functools.partial(
        pl.kernel,
        out_type=jax.ShapeDtypeStruct((nchunk, N_SC, FC), jnp.float32),
        mesh=mesh,
        scratch_types=[
            pltpu.VMEM((NSLOT, EBA), jnp.int32),
            pltpu.VMEM((NSLOT, EBA), jnp.int32),
            pltpu.VMEM((NSLOT, EBA, FC), jnp.float32),
            pltpu.VMEM_SHARED((N_SC, FC), jnp.float32),
            pltpu.SemaphoreType.DMA,
            pltpu.SemaphoreType.DMA,
            pltpu.SemaphoreType.DMA,
        ],
    )
    def agg_kernel(y_hbm, src_hbm, dst_hbm, z_hbm, s_hbm,
                   srcb, dstb, rows, acc, isem, gsem, ssem):
        cid = lax.axis_index("c")
        sid = lax.axis_index("s")
        lo = sid * ROWS_PT
        base0 = sid * ept

        def idx_load(k, b):
            pltpu.async_copy(src_hbm.at[pl.ds(base0 + k * EBA, EBA)],
                             srcb.at[b], isem)
            pltpu.async_copy(dst_hbm.at[pl.ds(base0 + k * EBA, EBA)],
                             dstb.at[b], isem)

        def wait_idx(k, b):
            pltpu.make_async_copy(src_hbm.at[pl.ds(base0 + k * EBA, EBA)],
                                  srcb.at[b], isem).wait()
            pltpu.make_async_copy(dst_hbm.at[pl.ds(base0 + k * EBA, EBA)],
                                  dstb.at[b], isem).wait()

        def gather(c, k, b):
            pltpu.async_copy(y_hbm.at[c].at[srcb.at[b]], rows.at[b], gsem)

        def wait_gather(c, b):
            pltpu.make_async_copy(y_hbm.at[c].at[srcb.at[b]],
                                  rows.at[b], gsem).wait()

        def scatter(k, b):
            pltpu.async_copy(rows.at[b], acc.at[dstb.at[b]], ssem, add=True)

        def wait_scatter(b):
            pltpu.make_async_copy(rows.at[b], acc.at[dstb.at[b]],
                                  ssem).wait()

        for kc in range(cpc):
            c = cid * cpc + kc
            pltpu.sync_copy(z_hbm, acc.at[pl.ds(lo, ROWS_PT)])
            plsc.subcore_barrier()

            for b in range(NSLOT):
                idx_load(b, b)
            for b in range(NSLOT):
                wait_idx(b, b)
                gather(c, b, b)

            @pl.loop(0, nblk - 1)
            def _(j):
                base = j * NSLOT
                for b in range(NSLOT):
                    wait_gather(c, b)
                    scatter(base + b, b)
                for b in range(NSLOT):
                    wait_scatter(b)
                    idx_load(base + NSLOT + b, b)
                for b in range(NSLOT):
                    wait_idx(base + NSLOT + b, b)
                    gather(c, base + NSLOT + b, b)

            for b in range(NSLOT):
                wait_gather(c, b)
                scatter((nblk - 1) * NSLOT + b, b)
            for b in range(NSLOT):
                wait_scatter(b)
            for r in range(rem):
                idx_load(nblk * NSLOT + r, r)
                wait_idx(nblk * NSLOT + r, r)
                pltpu.sync_copy(y_hbm.at[c].at[srcb.at[r]], rows.at[r])
                pltpu.sync_copy(rows.at[r], acc.at[dstb.at[r]], add=True)

            plsc.subcore_barrier()
            pltpu.sync_copy(acc.at[pl.ds(lo, ROWS_PT)],
                            s_hbm.at[c].at[pl.ds(lo, ROWS_PT)])

    return agg_kernel(y, src, dst, zeros128)


_RB = 2000


def _tc_xw1(x, w1r):
    nck = H1 // FC

    def body(x_ref, w_ref, y_ref):
        y_ref[0] = jnp.dot(x_ref[...], w_ref[0],
                           preferred_element_type=jnp.float32)

    return pl.pallas_call(
        body,
        grid=(nck, N // _RB),
        in_specs=[
            pl.BlockSpec((_RB, F_IN), lambda c, i: (i, 0)),
            pl.BlockSpec((1, F_IN, FC), lambda c, i: (c, 0, 0)),
        ],
        out_specs=pl.BlockSpec((1, _RB, FC), lambda c, i: (c, i, 0)),
        out_shape=jax.ShapeDtypeStruct((nck, N, FC), jnp.float32),
    )(x, w1r)


def _tc_scale_y1(xw, deg2):
    nck = H1 // FC

    def body(xw_ref, d_ref, y_ref, dinv_ref):
        dg = d_ref[0, :, 0:1] + d_ref[1, :, 0:1] + 1.0
        dinv = lax.rsqrt(dg)
        y_ref[0] = dinv * xw_ref[0]
        dinv_ref[...] = jnp.broadcast_to(dinv, (_RB, FC))

    return pl.pallas_call(
        body,
        grid=(nck, N // _RB),
        in_specs=[
            pl.BlockSpec((1, _RB, FC), lambda c, i: (c, i, 0)),
            pl.BlockSpec((NCORE, _RB, FC), lambda c, i: (0, i, 0)),
        ],
        out_specs=[
            pl.BlockSpec((1, _RB, FC), lambda c, i: (c, i, 0)),
            pl.BlockSpec((_RB, FC), lambda c, i: (i, 0)),
        ],
        out_shape=[
            jax.ShapeDtypeStruct((nck, N, FC), jnp.float32),
            jax.ShapeDtypeStruct((N, FC), jnp.float32),
        ],
    )(xw, deg2)


def _tc_z_stats(nchunk, y, s, dinv, br):
    h = nchunk * FC

    def body(y_ref, s_ref, d_ref, b_ref, z_ref, sum_ref, sq_ref):
        i = pl.program_id(1)
        zc = d_ref[:, 0:1] * (y_ref[0] + s_ref[0]) + b_ref[0]
        z_ref[...] = zc
        row0 = lax.broadcasted_iota(jnp.int32, (8, FC), 0) == 0

        @pl.when(i == 0)
        def _():
            sum_ref[...] = jnp.zeros((8, FC), jnp.float32)
            sq_ref[...] = jnp.zeros((8, FC), jnp.float32)

        sum_ref[...] += jnp.where(row0, jnp.sum(zc, 0)[None, :], 0.0)
        sq_ref[...] += jnp.where(row0, jnp.sum(zc * zc, 0)[None, :], 0.0)

    return pl.pallas_call(
        body,
        grid=(nchunk, N // _RB),
        in_specs=[
            pl.BlockSpec((1, _RB, FC), lambda c, i: (c, i, 0)),
            pl.BlockSpec((1, _RB, FC), lambda c, i: (c, i, 0)),
            pl.BlockSpec((_RB, FC), lambda c, i: (i, 0)),
            pl.BlockSpec((1, 1, FC), lambda c, i: (c, 0, 0)),
        ],
        out_specs=[
            pl.BlockSpec((_RB, FC), lambda c, i: (i, c)),
            pl.BlockSpec((8, FC), lambda c, i: (0, c)),
            pl.BlockSpec((8, FC), lambda c, i: (0, c)),
        ],
        out_shape=[
            jax.ShapeDtypeStruct((N, h), jnp.float32),
            jax.ShapeDtypeStruct((8, h), jnp.float32),
            jax.ShapeDtypeStruct((8, h), jnp.float32),
        ],
    )(y, s, dinv, br)


def _tc_bn_mm2(z1, ssum, ssq, g, b, w2r, dinv):
    nck = H2 // FC

    def body(z_ref, sum_ref, sq_ref, g_ref, b_ref, w_ref, d_ref, y_ref):
        mu = sum_ref[0] * (1.0 / N)
        var = sq_ref[0] * (1.0 / N) - mu * mu
        scale = g_ref[0, 0] * lax.rsqrt(var + EPS)
        shift = b_ref[0, 0] - mu * scale
        h = jnp.maximum(z_ref[...] * scale[None, :] + shift[None, :], 0.0)
        dinv = d_ref[:, 0:1]
        for c in range(nck):
            y_ref[c] = dinv * jnp.dot(h, w_ref[c],
                                      preferred_element_type=jnp.float32)

    return pl.pallas_call(
        body,
        grid=(N // _RB,),
        in_specs=[
            pl.BlockSpec((_RB, H1), lambda i: (i, 0)),
            pl.BlockSpec((8, H1), lambda i: (0, 0)),
            pl.BlockSpec((8, H1), lambda i: (0, 0)),
            pl.BlockSpec((1, 1, H1), lambda i: (0, 0, 0)),
            pl.BlockSpec((1, 1, H1), lambda i: (0, 0, 0)),
            pl.BlockSpec((nck, H1, FC), lambda i: (0, 0, 0)),
            pl.BlockSpec((_RB, FC), lambda i: (i, 0)),
        ],
        out_specs=pl.BlockSpec((nck, _RB, FC), lambda i: (0, i, 0)),
        out_shape=jax.ShapeDtypeStruct((nck, N, FC), jnp.float32),
    )(z1, ssum, ssq, g, b, w2r, dinv)


_PB = 1000


def _tc_bn_ln_pool(z2, ssum, ssq, bn_g, bn_b, ln_g, ln_b, batch128, batch3):
    def body(z_ref, sum_ref, sq_ref, g_ref, b_ref, lg_ref, lb_ref,
             bv_ref, bs_ref, out_ref, pool_ref):
        i = pl.program_id(0)
        mu = sum_ref[0] * (1.0 / N)
        var = sq_ref[0] * (1.0 / N) - mu * mu
        scale = g_ref[0, 0] * lax.rsqrt(var + EPS)
        shift = b_ref[0, 0] - mu * scale
        h = jnp.maximum(z_ref[...] * scale[None, :] + shift[None, :], 0.0)
        m = jnp.mean(h, 1, keepdims=True)
        v = jnp.mean(h * h, 1, keepdims=True) - m * m
        hn = lg_ref[0, 0][None, :] * (h - m) * lax.rsqrt(v + EPS) \
            + lb_ref[0, 0][None, :]

        @pl.when(i == 0)
        def _():
            pool_ref[...] = jnp.full((G, H2), -jnp.inf, jnp.float32)

        bcol = bv_ref[:, 0:1]
        g_lo = bs_ref[0, 0, 0]
        g_hi = bs_ref[0, 0, _PB - 1]

        def upd(gi, carry):
            mask = bcol == gi
            part = jnp.max(jnp.where(mask, hn, -jnp.inf), axis=0,
                           keepdims=True)
            cur = pool_ref[pl.ds(gi, 1), :]
            pool_ref[pl.ds(gi, 1), :] = jnp.maximum(cur, part)
            return carry

        lax.fori_loop(g_lo, g_hi + 1, upd, 0)

        @pl.when(i == (N // _PB) - 1)
        def _():
            out_ref[...] = pool_ref[...]

    return pl.pallas_call(
        body,
        grid=(N // _PB,),
        in_specs=[
            pl.BlockSpec((_PB, H2), lambda i: (i, 0)),
            pl.BlockSpec((8, H2), lambda i: (0, 0)),
            pl.BlockSpec((8, H2), lambda i: (0, 0)),
            pl.BlockSpec((1, 1, H2), lambda i: (0, 0, 0)),
            pl.BlockSpec((1, 1, H2), lambda i: (0, 0, 0)),
            pl.BlockSpec((1, 1, H2), lambda i: (0, 0, 0)),
            pl.BlockSpec((1, 1, H2), lambda i: (0, 0, 0)),
            pl.BlockSpec((_PB, 128), lambda i: (i, 0)),
            pl.BlockSpec((1, 1, _PB), lambda i: (i, 0, 0),
                         memory_space=pltpu.SMEM),
        ],
        out_specs=pl.BlockSpec((G, H2), lambda i: (0, 0)),
        out_shape=jax.ShapeDtypeStruct((G, H2), jnp.float32),
        scratch_shapes=[pltpu.VMEM((G, H2), jnp.float32)],
    )(z2, ssum, ssq, bn_g, bn_b, ln_g, ln_b, batch128, batch3)


def _tc_head(pooled, wl1, bl1, ln_g, ln_b, wl2p, bl2p):
    def body(p_ref, w1_ref, b1_ref, lg_ref, lb_ref, w2_ref, b2_ref, o_ref):
        p = p_ref[...]
        p = jnp.where(p < -1e38, 0.0, p)
        a = jnp.dot(p, w1_ref[...], preferred_element_type=jnp.float32)
        a = jnp.maximum(a + b1_ref[0, 0][None, :], 0.0)
        m = jnp.mean(a, 1, keepdims=True)
        v = jnp.mean(a * a, 1, keepdims=True) - m * m
        an = lg_ref[0, 0][None, :] * (a - m) * lax.rsqrt(v + EPS) \
            + lb_ref[0, 0][None, :]
        lg = jnp.dot(an, w2_ref[...], preferred_element_type=jnp.float32) \
            + b2_ref[0, 0][None, :]
        col = lax.broadcasted_iota(jnp.int32, (G, 128), 1)
        valid = col < C
        mx = jnp.max(jnp.where(valid, lg, -jnp.inf), 1, keepdims=True)
        ex = jnp.where(valid, jnp.exp(lg - mx), 0.0)
        lse = jnp.log(jnp.sum(ex, 1, keepdims=True)) + mx
        o_ref[...] = lg - lse

    return pl.pallas_call(
        body,
        grid=(1,),
        in_specs=[
            pl.BlockSpec((G, H2), lambda i: (0, 0)),
            pl.BlockSpec((H2, H1), lambda i: (0, 0)),
            pl.BlockSpec((1, 1, H1), lambda i: (0, 0, 0)),
            pl.BlockSpec((1, 1, H1), lambda i: (0, 0, 0)),
            pl.BlockSpec((1, 1, H1), lambda i: (0, 0, 0)),
            pl.BlockSpec((H1, 128), lambda i: (0, 0)),
            pl.BlockSpec((1, 1, 128), lambda i: (0, 0, 0)),
        ],
        out_specs=pl.BlockSpec((G, 128), lambda i: (0, 0)),
        out_shape=jax.ShapeDtypeStruct((G, 128), jnp.float32),
    )(pooled, wl1, bl1, ln_g, ln_b, wl2p, bl2p)


def kernel(x, edge_index, batch, W1, b1, bn1_g, bn1_b, W2, b2, bn2_g, bn2_b,
           ln1_g, ln1_b, Wl1, bl1, ln2_g, ln2_b, Wl2, bl2):
    src = edge_index[0]
    dst = edge_index[1]

    zeros128 = jnp.zeros((ROWS_PT, FC), jnp.float32)
    ones128 = jnp.ones((EB, FC), jnp.float32)

    w1r = W1.reshape(F_IN, H1 // FC, FC).transpose(1, 0, 2)
    w2r = W2.reshape(H1, H2 // FC, FC).transpose(1, 0, 2)
    b1r = b1.reshape(H1 // FC, 1, FC)
    b2r = b2.reshape(H2 // FC, 1, FC)
    batch3 = batch.reshape(N // _PB, 1, _PB)
    batch128 = jnp.broadcast_to(batch[:, None], (N, 128))
    wl2p = jnp.pad(Wl2, ((0, 0), (0, 128 - C)))
    bl2p = jnp.pad(bl2, (0, 128 - C)).reshape(1, 1, 128)

    deg2 = _sc_degree(dst, zeros128, ones128)
    xw1 = _tc_xw1(x, w1r)
    y1, dinv = _tc_scale_y1(xw1, deg2)
    s1 = _sc_aggregate(H1 // FC, y1, src, dst, zeros128)
    z1, sum1, sq1 = _tc_z_stats(H1 // FC, y1, s1, dinv, b1r)

    y2 = _tc_bn_mm2(z1, sum1, sq1, bn1_g.reshape(1, 1, H1),
                    bn1_b.reshape(1, 1, H1), w2r, dinv)
    s2 = _sc_aggregate(H2 // FC, y2, src, dst, zeros128)
    z2, sum2, sq2 = _tc_z_stats(H2 // FC, y2, s2, dinv, b2r)

    pooled = _tc_bn_ln_pool(z2, sum2, sq2, bn2_g.reshape(1, 1, H2),
                            bn2_b.reshape(1, 1, H2), ln1_g.reshape(1, 1, H2),
                            ln1_b.reshape(1, 1, H2), batch128, batch3)

    out = _tc_head(pooled, Wl1, bl1.reshape(1, 1, H1),
                   ln2_g.reshape(1, 1, H1), ln2_b.reshape(1, 1, H1),
                   wl2p, bl2p)
    return out[:, :C]


# speedup vs baseline: 1.1542x; 1.1542x over previous
import functools

import jax
import jax.numpy as jnp
from jax import lax
from jax.experimental import pallas as pl
from jax.experimental.pallas import tpu as pltpu
from jax.experimental.pallas import tpu_sc as plsc

N = 10000
E = 320000
F_IN = 128
H1 = 512
H2 = 1024
G = 128
C = 20

FC = 128
NCORE = 2
NSUB = 16
N_SC = 10112
ROWS_PT = N_SC // NSUB
EB = 80
EPS = 1e-5


def _sc_degree(dst, zeros128, ones128):
    mesh = plsc.VectorSubcoreMesh(core_axis_name="c", subcore_axis_name="s",
                                  num_cores=NCORE, num_subcores=NSUB)
    ept = E // (NCORE * NSUB)
    nb = ept // EB

    @functools.partial(
        pl.kernel,
        out_type=jax.ShapeDtypeStruct((NCORE, N_SC, FC), jnp.float32),
        mesh=mesh,
        scratch_types=[
            pltpu.VMEM((2, EB), jnp.int32),
            pltpu.VMEM((EB, FC), jnp.float32),
            pltpu.VMEM_SHARED((N_SC, FC), jnp.float32),
            pltpu.SemaphoreType.DMA,
            pltpu.SemaphoreType.DMA,
        ],
    )
    def deg_kernel(dst_hbm, z16_hbm, ones_hbm, deg_hbm, dst_v, ones_v,
                   acc, isem, ssem):
        cid = lax.axis_index("c")
        sid = lax.axis_index("s")
        lo = sid * ROWS_PT
        base0 = (cid * NSUB + sid) * ept
        nblk = nb // 2
        rem = nb - nblk * 2

        def idx_load(k, b):
            pltpu.async_copy(dst_hbm.at[pl.ds(base0 + k * EB, EB)],
                             dst_v.at[b], isem)

        def wait_idx(k, b):
            pltpu.make_async_copy(dst_hbm.at[pl.ds(base0 + k * EB, EB)],
                                  dst_v.at[b], isem).wait()

        def scat(b):
            pltpu.async_copy(ones_v, acc.at[dst_v.at[b]], ssem, add=True)

        def wait_scat(b):
            pltpu.make_async_copy(ones_v, acc.at[dst_v.at[b]], ssem).wait()

        pltpu.sync_copy(ones_hbm, ones_v)
        pltpu.sync_copy(z16_hbm, acc.at[pl.ds(lo, ROWS_PT)])
        plsc.subcore_barrier()

        for b in range(2):
            idx_load(b, b)

        @pl.loop(0, nblk - 1)
        def _(j):
            base = j * 2
            for b in range(2):
                wait_idx(base + b, b)
                scat(b)
            for b in range(2):
                wait_scat(b)
                idx_load(base + 2 + b, b)

        for b in range(2):
            wait_idx((nblk - 1) * 2 + b, b)
            scat(b)
        for b in range(2):
            wait_scat(b)
        for r in range(rem):
            k = nblk * 2 + r
            idx_load(k, r)
            wait_idx(k, r)
            pltpu.sync_copy(ones_v, acc.at[dst_v.at[r]], add=True)

        plsc.subcore_barrier()
        pltpu.sync_copy(acc.at[pl.ds(lo, ROWS_PT)],
                        deg_hbm.at[cid].at[pl.ds(lo, ROWS_PT)])

    return deg_kernel(dst, zeros128, ones128)


EBA = 80
NSLOT = 4


def _sc_aggregate(nchunk, y, src, dst, zeros128):
    mesh = plsc.VectorSubcoreMesh(core_axis_name="c", subcore_axis_name="s",
                                  num_cores=NCORE, num_subcores=NSUB)
    cpc = nchunk // NCORE
    ept = -(-E // (NSUB * EBA)) * EBA
    pad = ept * NSUB - E
    src = jnp.concatenate([src, jnp.zeros((pad,), jnp.int32)])
    dst = jnp.concatenate([dst, jnp.full((pad,), N, jnp.int32)])
    nbt = ept // EBA
    nblk = nbt // NSLOT
    rem = nbt - nblk * NSLOT

    @functools.partial(
        pl.kernel,
        out_type=jax.ShapeDtypeStruct((nchunk, N_SC, FC), jnp.float32),
        mesh=mesh,
        scratch_types=[
            pltpu.VMEM((NSLOT, EBA), jnp.int32),
            pltpu.VMEM((NSLOT, EBA), jnp.int32),
            pltpu.VMEM((NSLOT, EBA, FC), jnp.float32),
            pltpu.VMEM_SHARED((N_SC, FC), jnp.float32),
            pltpu.SemaphoreType.DMA,
            pltpu.SemaphoreType.DMA,
            pltpu.SemaphoreType.DMA,
        ],
    )
    def agg_kernel(y_hbm, src_hbm, dst_hbm, z_hbm, s_hbm,
                   srcb, dstb, rows, acc, isem, gsem, ssem):
        cid = lax.axis_index("c")
        sid = lax.axis_index("s")
        lo = sid * ROWS_PT
        base0 = sid * ept

        def idx_load(k, b):
            pltpu.async_copy(src_hbm.at[pl.ds(base0 + k * EBA, EBA)],
                             srcb.at[b], isem)
            pltpu.async_copy(dst_hbm.at[pl.ds(base0 + k * EBA, EBA)],
                             dstb.at[b], isem)

        def wait_idx(k, b):
            pltpu.make_async_copy(src_hbm.at[pl.ds(base0 + k * EBA, EBA)],
                                  srcb.at[b], isem).wait()
            pltpu.make_async_copy(dst_hbm.at[pl.ds(base0 + k * EBA, EBA)],
                                  dstb.at[b], isem).wait()

        def gather(c, k, b):
            pltpu.async_copy(y_hbm.at[c].at[srcb.at[b]], rows.at[b], gsem)

        def wait_gather(c, b):
            pltpu.make_async_copy(y_hbm.at[c].at[srcb.at[b]],
                                  rows.at[b], gsem).wait()

        def scatter(k, b):
            pltpu.async_copy(rows.at[b], acc.at[dstb.at[b]], ssem, add=True)

        def wait_scatter(b):
            pltpu.make_async_copy(rows.at[b], acc.at[dstb.at[b]],
                                  ssem).wait()

        for kc in range(cpc):
            c = cid * cpc + kc
            pltpu.sync_copy(z_hbm, acc.at[pl.ds(lo, ROWS_PT)])
            plsc.subcore_barrier()

            for b in range(NSLOT):
                idx_load(b, b)
            for b in range(NSLOT):
                wait_idx(b, b)
                gather(c, b, b)

            @pl.loop(0, nblk - 1)
            def _(j):
                base = j * NSLOT
                for b in range(NSLOT):
                    wait_gather(c, b)
                    scatter(base + b, b)
                for b in range(NSLOT):
                    wait_scatter(b)
                    idx_load(base + NSLOT + b, b)
                    wait_idx(base + NSLOT + b, b)
                    gather(c, base + NSLOT + b, b)

            for b in range(NSLOT):
                wait_gather(c, b)
                scatter((nblk - 1) * NSLOT + b, b)
            for b in range(NSLOT):
                wait_scatter(b)
            for r in range(rem):
                idx_load(nblk * NSLOT + r, r)
                wait_idx(nblk * NSLOT + r, r)
                pltpu.sync_copy(y_hbm.at[c].at[srcb.at[r]], rows.at[r])
                pltpu.sync_copy(rows.at[r], acc.at[dstb.at[r]], add=True)

            plsc.subcore_barrier()
            pltpu.sync_copy(acc.at[pl.ds(lo, ROWS_PT)],
                            s_hbm.at[c].at[pl.ds(lo, ROWS_PT)])

    return agg_kernel(y, src, dst, zeros128)


_RB = 2000


def _tc_xw1(x, w1r):
    nck = H1 // FC

    def body(x_ref, w_ref, y_ref):
        y_ref[0] = jnp.dot(x_ref[...], w_ref[0],
                           preferred_element_type=jnp.float32)

    return pl.pallas_call(
        body,
        grid=(nck, N // _RB),
        in_specs=[
            pl.BlockSpec((_RB, F_IN), lambda c, i: (i, 0)),
            pl.BlockSpec((1, F_IN, FC), lambda c, i: (c, 0, 0)),
        ],
        out_specs=pl.BlockSpec((1, _RB, FC), lambda c, i: (c, i, 0)),
        out_shape=jax.ShapeDtypeStruct((nck, N, FC), jnp.float32),
    )(x, w1r)


def _tc_scale_y1(xw, deg2):
    nck = H1 // FC

    def body(xw_ref, d_ref, y_ref, dinv_ref):
        dg = d_ref[0, :, 0:1] + d_ref[1, :, 0:1] + 1.0
        dinv = lax.rsqrt(dg)
        y_ref[0] = dinv * xw_ref[0]
        dinv_ref[...] = jnp.broadcast_to(dinv, (_RB, FC))

    return pl.pallas_call(
        body,
        grid=(nck, N // _RB),
        in_specs=[
            pl.BlockSpec((1, _RB, FC), lambda c, i: (c, i, 0)),
            pl.BlockSpec((NCORE, _RB, FC), lambda c, i: (0, i, 0)),
        ],
        out_specs=[
            pl.BlockSpec((1, _RB, FC), lambda c, i: (c, i, 0)),
            pl.BlockSpec((_RB, FC), lambda c, i: (i, 0)),
        ],
        out_shape=[
            jax.ShapeDtypeStruct((nck, N, FC), jnp.float32),
            jax.ShapeDtypeStruct((N, FC), jnp.float32),
        ],
    )(xw, deg2)


def _tc_z_stats(nchunk, y, s, dinv, br):
    h = nchunk * FC

    def body(y_ref, s_ref, d_ref, b_ref, z_ref, sum_ref, sq_ref):
        i = pl.program_id(1)
        zc = d_ref[:, 0:1] * (y_ref[0] + s_ref[0]) + b_ref[0]
        z_ref[...] = zc
        row0 = lax.broadcasted_iota(jnp.int32, (8, FC), 0) == 0

        @pl.when(i == 0)
        def _():
            sum_ref[...] = jnp.zeros((8, FC), jnp.float32)
            sq_ref[...] = jnp.zeros((8, FC), jnp.float32)

        sum_ref[...] += jnp.where(row0, jnp.sum(zc, 0)[None, :], 0.0)
        sq_ref[...] += jnp.where(row0, jnp.sum(zc * zc, 0)[None, :], 0.0)

    return pl.pallas_call(
        body,
        grid=(nchunk, N // _RB),
        in_specs=[
            pl.BlockSpec((1, _RB, FC), lambda c, i: (c, i, 0)),
            pl.BlockSpec((1, _RB, FC), lambda c, i: (c, i, 0)),
            pl.BlockSpec((_RB, FC), lambda c, i: (i, 0)),
            pl.BlockSpec((1, 1, FC), lambda c, i: (c, 0, 0)),
        ],
        out_specs=[
            pl.BlockSpec((_RB, FC), lambda c, i: (i, c)),
            pl.BlockSpec((8, FC), lambda c, i: (0, c)),
            pl.BlockSpec((8, FC), lambda c, i: (0, c)),
        ],
        out_shape=[
            jax.ShapeDtypeStruct((N, h), jnp.float32),
            jax.ShapeDtypeStruct((8, h), jnp.float32),
            jax.ShapeDtypeStruct((8, h), jnp.float32),
        ],
    )(y, s, dinv, br)


def _tc_bn_mm2(z1, ssum, ssq, g, b, w2r, dinv):
    nck = H2 // FC

    def body(z_ref, sum_ref, sq_ref, g_ref, b_ref, w_ref, d_ref, y_ref):
        mu = sum_ref[0] * (1.0 / N)
        var = sq_ref[0] * (1.0 / N) - mu * mu
        scale = g_ref[0, 0] * lax.rsqrt(var + EPS)
        shift = b_ref[0, 0] - mu * scale
        h = jnp.maximum(z_ref[...] * scale[None, :] + shift[None, :], 0.0)
        dinv = d_ref[:, 0:1]
        for c in range(nck):
            y_ref[c] = dinv * jnp.dot(h, w_ref[c],
                                      preferred_element_type=jnp.float32)

    return pl.pallas_call(
        body,
        grid=(N // _RB,),
        in_specs=[
            pl.BlockSpec((_RB, H1), lambda i: (i, 0)),
            pl.BlockSpec((8, H1), lambda i: (0, 0)),
            pl.BlockSpec((8, H1), lambda i: (0, 0)),
            pl.BlockSpec((1, 1, H1), lambda i: (0, 0, 0)),
            pl.BlockSpec((1, 1, H1), lambda i: (0, 0, 0)),
            pl.BlockSpec((nck, H1, FC), lambda i: (0, 0, 0)),
            pl.BlockSpec((_RB, FC), lambda i: (i, 0)),
        ],
        out_specs=pl.BlockSpec((nck, _RB, FC), lambda i: (0, i, 0)),
        out_shape=jax.ShapeDtypeStruct((nck, N, FC), jnp.float32),
    )(z1, ssum, ssq, g, b, w2r, dinv)


_PB = 1000


def _tc_bn_ln_pool(z2, ssum, ssq, bn_g, bn_b, ln_g, ln_b, batch128, batch3):
    def body(z_ref, sum_ref, sq_ref, g_ref, b_ref, lg_ref, lb_ref,
             bv_ref, bs_ref, out_ref, pool_ref):
        i = pl.program_id(0)
        mu = sum_ref[0] * (1.0 / N)
        var = sq_ref[0] * (1.0 / N) - mu * mu
        scale = g_ref[0, 0] * lax.rsqrt(var + EPS)
        shift = b_ref[0, 0] - mu * scale
        h = jnp.maximum(z_ref[...] * scale[None, :] + shift[None, :], 0.0)
        m = jnp.mean(h, 1, keepdims=True)
        v = jnp.mean(h * h, 1, keepdims=True) - m * m
        hn = lg_ref[0, 0][None, :] * (h - m) * lax.rsqrt(v + EPS) \
            + lb_ref[0, 0][None, :]

        @pl.when(i == 0)
        def _():
            pool_ref[...] = jnp.full((G, H2), -jnp.inf, jnp.float32)

        bcol = bv_ref[:, 0:1]
        g_lo = bs_ref[0, 0, 0]
        g_hi = bs_ref[0, 0, _PB - 1]

        def upd(gi, carry):
            mask = bcol == gi
            part = jnp.max(jnp.where(mask, hn, -jnp.inf), axis=0,
                           keepdims=True)
            cur = pool_ref[pl.ds(gi, 1), :]
            pool_ref[pl.ds(gi, 1), :] = jnp.maximum(cur, part)
            return carry

        lax.fori_loop(g_lo, g_hi + 1, upd, 0)

        @pl.when(i == (N // _PB) - 1)
        def _():
            out_ref[...] = pool_ref[...]

    return pl.pallas_call(
        body,
        grid=(N // _PB,),
        in_specs=[
            pl.BlockSpec((_PB, H2), lambda i: (i, 0)),
            pl.BlockSpec((8, H2), lambda i: (0, 0)),
            pl.BlockSpec((8, H2), lambda i: (0, 0)),
            pl.BlockSpec((1, 1, H2), lambda i: (0, 0, 0)),
            pl.BlockSpec((1, 1, H2), lambda i: (0, 0, 0)),
            pl.BlockSpec((1, 1, H2), lambda i: (0, 0, 0)),
            pl.BlockSpec((1, 1, H2), lambda i: (0, 0, 0)),
            pl.BlockSpec((_PB, 128), lambda i: (i, 0)),
            pl.BlockSpec((1, 1, _PB), lambda i: (i, 0, 0),
                         memory_space=pltpu.SMEM),
        ],
        out_specs=pl.BlockSpec((G, H2), lambda i: (0, 0)),
        out_shape=jax.ShapeDtypeStruct((G, H2), jnp.float32),
        scratch_shapes=[pltpu.VMEM((G, H2), jnp.float32)],
    )(z2, ssum, ssq, bn_g, bn_b, ln_g, ln_b, batch128, batch3)


def _tc_head(pooled, wl1, bl1, ln_g, ln_b, wl2p, bl2p):
    def body(p_ref, w1_ref, b1_ref, lg_ref, lb_ref, w2_ref, b2_ref, o_ref):
        p = p_ref[...]
        p = jnp.where(p < -1e38, 0.0, p)
        a = jnp.dot(p, w1_ref[...], preferred_element_type=jnp.float32)
        a = jnp.maximum(a + b1_ref[0, 0][None, :], 0.0)
        m = jnp.mean(a, 1, keepdims=True)
        v = jnp.mean(a * a, 1, keepdims=True) - m * m
        an = lg_ref[0, 0][None, :] * (a - m) * lax.rsqrt(v + EPS) \
            + lb_ref[0, 0][None, :]
        lg = jnp.dot(an, w2_ref[...], preferred_element_type=jnp.float32) \
            + b2_ref[0, 0][None, :]
        col = lax.broadcasted_iota(jnp.int32, (G, 128), 1)
        valid = col < C
        mx = jnp.max(jnp.where(valid, lg, -jnp.inf), 1, keepdims=True)
        ex = jnp.where(valid, jnp.exp(lg - mx), 0.0)
        lse = jnp.log(jnp.sum(ex, 1, keepdims=True)) + mx
        o_ref[...] = lg - lse

    return pl.pallas_call(
        body,
        grid=(1,),
        in_specs=[
            pl.BlockSpec((G, H2), lambda i: (0, 0)),
            pl.BlockSpec((H2, H1), lambda i: (0, 0)),
            pl.BlockSpec((1, 1, H1), lambda i: (0, 0, 0)),
            pl.BlockSpec((1, 1, H1), lambda i: (0, 0, 0)),
            pl.BlockSpec((1, 1, H1), lambda i: (0, 0, 0)),
            pl.BlockSpec((H1, 128), lambda i: (0, 0)),
            pl.BlockSpec((1, 1, 128), lambda i: (0, 0, 0)),
        ],
        out_specs=pl.BlockSpec((G, 128), lambda i: (0, 0)),
        out_shape=jax.ShapeDtypeStruct((G, 128), jnp.float32),
    )(pooled, wl1, bl1, ln_g, ln_b, wl2p, bl2p)


def kernel(x, edge_index, batch, W1, b1, bn1_g, bn1_b, W2, b2, bn2_g, bn2_b,
           ln1_g, ln1_b, Wl1, bl1, ln2_g, ln2_b, Wl2, bl2):
    src = edge_index[0]
    dst = edge_index[1]

    zeros128 = jnp.zeros((ROWS_PT, FC), jnp.float32)
    ones128 = jnp.ones((EB, FC), jnp.float32)

    w1r = W1.reshape(F_IN, H1 // FC, FC).transpose(1, 0, 2)
    w2r = W2.reshape(H1, H2 // FC, FC).transpose(1, 0, 2)
    b1r = b1.reshape(H1 // FC, 1, FC)
    b2r = b2.reshape(H2 // FC, 1, FC)
    batch3 = batch.reshape(N // _PB, 1, _PB)
    batch128 = jnp.broadcast_to(batch[:, None], (N, 128))
    wl2p = jnp.pad(Wl2, ((0, 0), (0, 128 - C)))
    bl2p = jnp.pad(bl2, (0, 128 - C)).reshape(1, 1, 128)

    deg2 = _sc_degree(dst, zeros128, ones128)
    xw1 = _tc_xw1(x, w1r)
    y1, dinv = _tc_scale_y1(xw1, deg2)
    s1 = _sc_aggregate(H1 // FC, y1, src, dst, zeros128)
    z1, sum1, sq1 = _tc_z_stats(H1 // FC, y1, s1, dinv, b1r)

    y2 = _tc_bn_mm2(z1, sum1, sq1, bn1_g.reshape(1, 1, H1),
                    bn1_b.reshape(1, 1, H1), w2r, dinv)
    s2 = _sc_aggregate(H2 // FC, y2, src, dst, zeros128)
    z2, sum2, sq2 = _tc_z_stats(H2 // FC, y2, s2, dinv, b2r)

    pooled = _tc_bn_ln_pool(z2, sum2, sq2, bn2_g.reshape(1, 1, H2),
                            bn2_b.reshape(1, 1, H2), ln1_g.reshape(1, 1, H2),
                            ln1_b.reshape(1, 1, H2), batch128, batch3)

    out = _tc_head(pooled, Wl1, bl1.reshape(1, 1, H1),
                   ln2_g.reshape(1, 1, H1), ln2_b.reshape(1, 1, H1),
                   wl2p, bl2p)
    return out[:, :C]
